# Initial kernel scaffold; baseline (speedup 1.0000x reference)
import jax
import jax.numpy as jnp
from jax.experimental import pallas as pl


def kernel(indices, embs):
    raise NotImplementedError("write your pallas kernel here")


# speedup vs baseline: 4.5620x; 4.5620x over previous
import functools

import jax
import jax.numpy as jnp
from jax import lax
from jax.experimental import pallas as pl
from jax.experimental.pallas import tpu as pltpu
from jax.experimental.pallas import tpu_sc as plsc

HDIM = 32
B_TOTAL = 16384 * 200
_NC, _NS = 2, 16
NW = _NC * _NS
B_PER_W = B_TOTAL // NW
CHUNK = 512
N_CHUNKS = B_PER_W // CHUNK

_mesh = plsc.VectorSubcoreMesh(core_axis_name="c", subcore_axis_name="s")


@functools.partial(
    pl.kernel,
    mesh=_mesh,
    out_type=jax.ShapeDtypeStruct((B_TOTAL, HDIM), jnp.float32),
    scratch_types=[
        pltpu.VMEM((CHUNK,), jnp.int32),
        pltpu.VMEM((CHUNK, HDIM), jnp.float32),
        pltpu.SemaphoreType.DMA,
    ],
    compiler_params=pltpu.CompilerParams(use_tc_tiling_on_sc=False),
)
def _gather_kernel(idx_hbm, table_hbm, out_hbm, idx_v, rows_v, sem):
    wid = lax.axis_index("s") * _NC + lax.axis_index("c")
    base = wid * B_PER_W

    def body(i, carry):
        off = base + i * CHUNK
        pltpu.sync_copy(idx_hbm.at[pl.ds(off, CHUNK)], idx_v)
        pltpu.async_copy(table_hbm.at[idx_v], rows_v, sem).wait()
        pltpu.sync_copy(rows_v, out_hbm.at[pl.ds(off, CHUNK)])
        return carry

    lax.fori_loop(0, N_CHUNKS, body, 0)


def kernel(indices, embs):
    flat_idx = indices.reshape(-1).astype(jnp.int32)
    out = _gather_kernel(flat_idx, embs)
    return out.reshape(indices.shape + (HDIM,))


# speedup vs baseline: 5.0492x; 1.1068x over previous
import functools

import jax
import jax.numpy as jnp
from jax import lax
from jax.experimental import pallas as pl
from jax.experimental.pallas import tpu as pltpu
from jax.experimental.pallas import tpu_sc as plsc

HDIM = 32
B_TOTAL = 16384 * 200
_NC, _NS = 2, 16
NW = _NC * _NS
B_PER_W = B_TOTAL // NW
CHUNK = 512
N_CHUNKS = B_PER_W // CHUNK
NBUF = 4
N_GROUPS = N_CHUNKS // NBUF

_mesh = plsc.VectorSubcoreMesh(core_axis_name="c", subcore_axis_name="s")


@functools.partial(
    pl.kernel,
    mesh=_mesh,
    out_type=jax.ShapeDtypeStruct((B_TOTAL, HDIM), jnp.float32),
    scratch_types=(
        [pltpu.VMEM((CHUNK,), jnp.int32) for _ in range(NBUF)]
        + [pltpu.VMEM((CHUNK, HDIM), jnp.float32) for _ in range(NBUF)]
        + [pltpu.SemaphoreType.DMA for _ in range(3 * NBUF)]
    ),
    compiler_params=pltpu.CompilerParams(use_tc_tiling_on_sc=False),
)
def _gather_kernel(idx_hbm, table_hbm, out_hbm, *scratch):
    idx_v = scratch[:NBUF]
    rows_v = scratch[NBUF:2 * NBUF]
    sem_i = scratch[2 * NBUF:3 * NBUF]
    sem_g = scratch[3 * NBUF:4 * NBUF]
    sem_s = scratch[4 * NBUF:5 * NBUF]

    wid = lax.axis_index("s") * _NC + lax.axis_index("c")
    base = wid * B_PER_W

    def start_idx(g, b):
        pltpu.async_copy(
            idx_hbm.at[pl.ds(base + g * CHUNK, CHUNK)], idx_v[b], sem_i[b])

    def wait_idx(b):
        pltpu.make_async_copy(
            idx_hbm.at[pl.ds(base, CHUNK)], idx_v[b], sem_i[b]).wait()

    def start_gather(b):
        pltpu.async_copy(table_hbm.at[idx_v[b]], rows_v[b], sem_g[b])

    def wait_gather(b):
        pltpu.make_async_copy(
            table_hbm.at[pl.ds(0, CHUNK)], rows_v[b], sem_g[b]).wait()

    def start_store(g, b):
        pltpu.async_copy(
            rows_v[b], out_hbm.at[pl.ds(base + g * CHUNK, CHUNK)], sem_s[b])

    def wait_store(b):
        pltpu.make_async_copy(
            rows_v[b], out_hbm.at[pl.ds(base, CHUNK)], sem_s[b]).wait()

    start_idx(0, 0)
    start_idx(1, 1)

    @pl.loop(0, N_GROUPS)
    def _group(t):
        for b in range(NBUF):
            g = t * NBUF + b
            @pl.when(t > 0)
            def _():
                wait_store(b)
            wait_idx(b)
            start_gather(b)
            @pl.when(g + 2 < N_CHUNKS)
            def _():
                start_idx(g + 2, (b + 2) % NBUF)
            @pl.when(g > 0)
            def _():
                bp = (b + NBUF - 1) % NBUF
                wait_gather(bp)
                start_store(g - 1, bp)

    bl = (N_CHUNKS - 1) % NBUF
    wait_gather(bl)
    start_store(N_CHUNKS - 1, bl)
    for b in range(NBUF):
        wait_store(b)


def kernel(indices, embs):
    flat_idx = indices.reshape(-1).astype(jnp.int32)
    out = _gather_kernel(flat_idx, embs)
    return out.reshape(indices.shape + (HDIM,))


# speedup vs baseline: 5.0517x; 1.0005x over previous
import functools

import jax
import jax.numpy as jnp
from jax import lax
from jax.experimental import pallas as pl
from jax.experimental.pallas import tpu as pltpu
from jax.experimental.pallas import tpu_sc as plsc

HDIM = 32
B_TOTAL = 16384 * 200
_NC, _NS = 2, 16
NW = _NC * _NS
B_PER_W = B_TOTAL // NW
CHUNK = 800
N_CHUNKS = B_PER_W // CHUNK
NBUF = 4
N_GROUPS = N_CHUNKS // NBUF

_mesh = plsc.VectorSubcoreMesh(core_axis_name="c", subcore_axis_name="s")


@functools.partial(
    pl.kernel,
    mesh=_mesh,
    out_type=jax.ShapeDtypeStruct((B_TOTAL, HDIM), jnp.float32),
    scratch_types=(
        [pltpu.VMEM((CHUNK,), jnp.int32) for _ in range(NBUF)]
        + [pltpu.VMEM((CHUNK, HDIM), jnp.float32) for _ in range(NBUF)]
        + [pltpu.SemaphoreType.DMA for _ in range(3 * NBUF)]
    ),
    compiler_params=pltpu.CompilerParams(use_tc_tiling_on_sc=False),
)
def _gather_kernel(idx_hbm, table_hbm, out_hbm, *scratch):
    idx_v = scratch[:NBUF]
    rows_v = scratch[NBUF:2 * NBUF]
    sem_i = scratch[2 * NBUF:3 * NBUF]
    sem_g = scratch[3 * NBUF:4 * NBUF]
    sem_s = scratch[4 * NBUF:5 * NBUF]

    wid = lax.axis_index("s") * _NC + lax.axis_index("c")
    base = wid * B_PER_W

    def start_idx(g, b):
        pltpu.async_copy(
            idx_hbm.at[pl.ds(base + g * CHUNK, CHUNK)], idx_v[b], sem_i[b])

    def wait_idx(b):
        pltpu.make_async_copy(
            idx_hbm.at[pl.ds(base, CHUNK)], idx_v[b], sem_i[b]).wait()

    def start_gather(b):
        pltpu.async_copy(table_hbm.at[idx_v[b]], rows_v[b], sem_g[b])

    def wait_gather(b):
        pltpu.make_async_copy(
            table_hbm.at[pl.ds(0, CHUNK)], rows_v[b], sem_g[b]).wait()

    def start_store(g, b):
        pltpu.async_copy(
            rows_v[b], out_hbm.at[pl.ds(base + g * CHUNK, CHUNK)], sem_s[b])

    def wait_store(b):
        pltpu.make_async_copy(
            rows_v[b], out_hbm.at[pl.ds(base, CHUNK)], sem_s[b]).wait()

    start_idx(0, 0)
    start_idx(1, 1)

    @pl.loop(0, N_GROUPS)
    def _group(t):
        for b in range(NBUF):
            g = t * NBUF + b
            @pl.when(t > 0)
            def _():
                wait_store(b)
            wait_idx(b)
            start_gather(b)
            @pl.when(g + 2 < N_CHUNKS)
            def _():
                start_idx(g + 2, (b + 2) % NBUF)
            @pl.when(g > 0)
            def _():
                bp = (b + NBUF - 1) % NBUF
                wait_gather(bp)
                start_store(g - 1, bp)

    bl = (N_CHUNKS - 1) % NBUF
    wait_gather(bl)
    start_store(N_CHUNKS - 1, bl)
    for b in range(NBUF):
        wait_store(b)


def kernel(indices, embs):
    flat_idx = indices.reshape(-1).astype(jnp.int32)
    out = _gather_kernel(flat_idx, embs)
    return out.reshape(indices.shape + (HDIM,))
